# Initial kernel scaffold; baseline (speedup 1.0000x reference)
import jax
import jax.numpy as jnp
from jax import lax
from jax.experimental import pallas as pl
from jax.experimental.pallas import tpu as pltpu
from jax.experimental.pallas import tpu_sc as plsc

N = 10000
NPAD = 10240
E = 320000
D_FEAT = 128
D_EDGE = 16
LATENT = 128
N_ACT = 16

NC = 2
NS = 16
NW = NC * NS
EPW = E // NW
B = 80
NB = EPW // B


def _mm_bias_kernel(x_ref, w_ref, b_ref, o_ref):
    o_ref[...] = (
        jnp.dot(x_ref[...], w_ref[...], preferred_element_type=jnp.float32)
        + b_ref[...]
    )


def _mm_bias(x, W, b, blk):
    M, K = x.shape
    Nout = W.shape[1]
    return pl.pallas_call(
        _mm_bias_kernel,
        grid=(M // blk,),
        in_specs=[
            pl.BlockSpec((blk, K), lambda i: (i, 0)),
            pl.BlockSpec((K, Nout), lambda i: (0, 0)),
            pl.BlockSpec((1, Nout), lambda i: (0, 0)),
        ],
        out_specs=pl.BlockSpec((blk, Nout), lambda i: (i, 0)),
        out_shape=jax.ShapeDtypeStruct((M, Nout), jnp.float32),
    )(x, W, b.reshape(1, -1))


def _combine_mm_kernel(acc_ref, den_ref, bias_ref, w_ref, b2_ref, o_ref):
    den = den_ref[0] + den_ref[1]
    inv = 1.0 / (den + 1e-16)
    lat = (acc_ref[0] + acc_ref[1]) * inv[:, None] + bias_ref[...]
    o_ref[...] = (
        jnp.dot(lat, w_ref[...], preferred_element_type=jnp.float32)
        + b2_ref[...]
    )


def _combine_mm(acc_p, den_p, bias, W, b2, blk=1024):
    D = acc_p.shape[2]
    Nout = W.shape[1]
    return pl.pallas_call(
        _combine_mm_kernel,
        grid=(NPAD // blk,),
        in_specs=[
            pl.BlockSpec((2, blk, D), lambda i: (0, i, 0)),
            pl.BlockSpec((2, blk), lambda i: (0, i)),
            pl.BlockSpec((1, D), lambda i: (0, 0)),
            pl.BlockSpec((D, Nout), lambda i: (0, 0)),
            pl.BlockSpec((1, Nout), lambda i: (0, 0)),
        ],
        out_specs=pl.BlockSpec((blk, Nout), lambda i: (i, 0)),
        out_shape=jax.ShapeDtypeStruct((NPAD, Nout), jnp.float32),
    )(acc_p, den_p, bias.reshape(1, -1), W, b2.reshape(1, -1))


def _final_kernel(acc_ref, den_ref, bias_ref, o_ref):
    den = den_ref[0] + den_ref[1]
    inv = 1.0 / (den + 1e-16)
    o_ref[...] = (acc_ref[0] + acc_ref[1]) * inv[:, None] + bias_ref[...]


def _final_combine(acc_p, den_p, bias, blk=2048):
    D = acc_p.shape[2]
    return pl.pallas_call(
        _final_kernel,
        grid=(NPAD // blk,),
        in_specs=[
            pl.BlockSpec((2, blk, D), lambda i: (0, i, 0)),
            pl.BlockSpec((2, blk), lambda i: (0, i)),
            pl.BlockSpec((1, D), lambda i: (0, 0)),
        ],
        out_specs=pl.BlockSpec((blk, D), lambda i: (i, 0)),
        out_shape=jax.ShapeDtypeStruct((NPAD, D), jnp.float32),
    )(acc_p, den_p, bias.reshape(1, -1))


def _make_gat_sc(D):
    mesh = plsc.VectorSubcoreMesh(
        core_axis_name="c", subcore_axis_name="s", num_cores=NC, num_subcores=NS
    )
    CH = D // 16
    RPT = NPAD // NS
    ZR = 128

    def body(xl_hbm, src_hbm, dst_hbm, ef_hbm, att_hbm,
             den_out, acc_out,
             src_v, dst_v, xs_v, xd_v, ef_v, ex_v, att_v, zrow_v, zden_v,
             den_s, acc_s, sem0, sem1, sem2):
        c = lax.axis_index("c")
        s = lax.axis_index("s")
        g = c * NS + s

        zero16 = jnp.zeros((16,), jnp.float32)

        def zden_body(i, _):
            zden_v[pl.ds(i * 16, 16)] = zero16
            return 0
        lax.fori_loop(0, RPT // 16, zden_body, 0)

        def zrow_body(i, _):
            for cc in range(CH):
                zrow_v[i, pl.ds(cc * 16, 16)] = zero16
            return 0
        lax.fori_loop(0, ZR, zrow_body, 0)

        pltpu.sync_copy(zden_v, den_s.at[pl.ds(s * RPT, RPT)])
        for j in range(RPT // ZR):
            pltpu.sync_copy(zrow_v, acc_s.at[pl.ds(s * RPT + j * ZR, ZR)])
        plsc.subcore_barrier()

        pltpu.sync_copy(att_hbm, att_v)
        att_c = [att_v[pl.ds(cc * 16, 16)] for cc in range(CH)]
        iota16 = lax.iota(jnp.int32, 16)

        def batch(t, _):
            base = g * EPW + t * B
            pltpu.sync_copy(src_hbm.at[pl.ds(base, B)], src_v)
            pltpu.sync_copy(dst_hbm.at[pl.ds(base, B)], dst_v)
            cp0 = pltpu.async_copy(xl_hbm.at[src_v], xs_v, sem0)
            cp1 = pltpu.async_copy(xl_hbm.at[dst_v], xd_v, sem1)
            cp2 = pltpu.async_copy(ef_hbm.at[pl.ds(base, B)], ef_v, sem2)
            cp0.wait()
            cp1.wait()
            cp2.wait()

            def group(gi, _):
                e0 = gi * 16
                eidx = e0 + iota16
                acc = jnp.zeros((16,), jnp.float32)
                for f in range(D):
                    cc, jj = f // 16, f % 16
                    colf = jnp.full((16,), f, jnp.int32)
                    vs = plsc.load_gather(xs_v, [eidx, colf])
                    vd = plsc.load_gather(xd_v, [eidx, colf])
                    ve = plsc.load_gather(ef_v, [eidx, colf])
                    sv = vs + vd + ve
                    lv = jnp.maximum(sv, 0.2 * sv)
                    acc = acc + lv * att_c[cc][jj]
                ex16 = jnp.exp(acc)
                ex_v[pl.ds(e0, 16)] = ex16
                for el in range(16):
                    exs = ex16[el]
                    e = e0 + el
                    for cc in range(CH):
                        xs_v[e, pl.ds(cc * 16, 16)] = (
                            xs_v[e, pl.ds(cc * 16, 16)] * exs
                        )
                return 0
            lax.fori_loop(0, B // 16, group, 0)

            pltpu.sync_copy(ex_v, den_s.at[dst_v], add=True)
            pltpu.sync_copy(xs_v, acc_s.at[dst_v], add=True)
            return 0
        lax.fori_loop(0, NB, batch, 0)

        plsc.subcore_barrier()
        pltpu.sync_copy(den_s.at[pl.ds(s * RPT, RPT)],
                        den_out.at[c, pl.ds(s * RPT, RPT)])
        pltpu.sync_copy(acc_s.at[pl.ds(s * RPT, RPT)],
                        acc_out.at[c, pl.ds(s * RPT, RPT)])

    return pl.kernel(
        body,
        out_type=(
            jax.ShapeDtypeStruct((NC, NPAD), jnp.float32),
            jax.ShapeDtypeStruct((NC, NPAD, D), jnp.float32),
        ),
        mesh=mesh,
        scratch_types=[
            pltpu.VMEM((B,), jnp.int32),
            pltpu.VMEM((B,), jnp.int32),
            pltpu.VMEM((B, D), jnp.float32),
            pltpu.VMEM((B, D), jnp.float32),
            pltpu.VMEM((B, D), jnp.float32),
            pltpu.VMEM((B,), jnp.float32),
            pltpu.VMEM((D,), jnp.float32),
            pltpu.VMEM((ZR, D), jnp.float32),
            pltpu.VMEM((RPT,), jnp.float32),
            pltpu.VMEM_SHARED((NPAD,), jnp.float32),
            pltpu.VMEM_SHARED((NPAD, D), jnp.float32),
            pltpu.SemaphoreType.DMA,
            pltpu.SemaphoreType.DMA,
            pltpu.SemaphoreType.DMA,
        ],
    )


_gat_sc_128 = _make_gat_sc(LATENT)
_gat_sc_16 = _make_gat_sc(N_ACT)


def kernel(x, edge_index, edge_attr,
           W1, b1, We1, att1, bias1,
           W2, b2, We2, att2, bias2):
    src = edge_index[0]
    dst = edge_index[1]
    zero128 = jnp.zeros((LATENT,), jnp.float32)
    zero16 = jnp.zeros((N_ACT,), jnp.float32)

    xl1 = _mm_bias(x, W1, b1, blk=2000)
    ef1 = _mm_bias(edge_attr, We1, zero128, blk=4000)
    den1, acc1 = _gat_sc_128(xl1, src, dst, ef1, att1)

    xl2 = _combine_mm(acc1, den1, bias1, W2, b2)
    ef2 = _mm_bias(edge_attr, We2, zero16, blk=4000)
    den2, acc2 = _gat_sc_16(xl2, src, dst, ef2, att2)

    action_logits = _final_combine(acc2, den2, bias2)[:N]

    flat = action_logits.reshape(-1)
    skey = jax.random.key(42)
    idx = jax.random.categorical(skey, flat)
    log_prob = jax.nn.log_softmax(flat)[idx]
    sel_node, sel_action = jnp.unravel_index(idx, action_logits.shape)
    return (sel_node, sel_action, log_prob)


# speedup vs baseline: 4.6811x; 4.6811x over previous
import jax
import jax.numpy as jnp
from jax import lax
from jax.experimental import pallas as pl
from jax.experimental.pallas import tpu as pltpu
from jax.experimental.pallas import tpu_sc as plsc

N = 10000
NPAD = 10240
E = 320000
D_FEAT = 128
D_EDGE = 16
LATENT = 128
N_ACT = 16

NC = 2
NS = 16
NW = NC * NS
EPW = E // NW
B = 80
NB = EPW // B


def _mm_bias_kernel(x_ref, w_ref, b_ref, o_ref):
    o_ref[...] = (
        jnp.dot(x_ref[...], w_ref[...], preferred_element_type=jnp.float32)
        + b_ref[...]
    )


def _mm_bias(x, W, b, blk):
    M, K = x.shape
    Nout = W.shape[1]
    return pl.pallas_call(
        _mm_bias_kernel,
        grid=(M // blk,),
        in_specs=[
            pl.BlockSpec((blk, K), lambda i: (i, 0)),
            pl.BlockSpec((K, Nout), lambda i: (0, 0)),
            pl.BlockSpec((1, Nout), lambda i: (0, 0)),
        ],
        out_specs=pl.BlockSpec((blk, Nout), lambda i: (i, 0)),
        out_shape=jax.ShapeDtypeStruct((M, Nout), jnp.float32),
    )(x, W, b.reshape(1, -1))


def _combine_mm_kernel(acc_ref, den_ref, bias_ref, w_ref, b2_ref, o_ref):
    den = den_ref[0] + den_ref[1]
    inv = 1.0 / (den + 1e-16)
    lat = (acc_ref[0] + acc_ref[1]) * inv[:, None] + bias_ref[...]
    o_ref[...] = (
        jnp.dot(lat, w_ref[...], preferred_element_type=jnp.float32)
        + b2_ref[...]
    )


def _combine_mm(acc_p, den_p, bias, W, b2, blk=1024):
    D = acc_p.shape[2]
    Nout = W.shape[1]
    return pl.pallas_call(
        _combine_mm_kernel,
        grid=(NPAD // blk,),
        in_specs=[
            pl.BlockSpec((2, blk, D), lambda i: (0, i, 0)),
            pl.BlockSpec((2, blk), lambda i: (0, i)),
            pl.BlockSpec((1, D), lambda i: (0, 0)),
            pl.BlockSpec((D, Nout), lambda i: (0, 0)),
            pl.BlockSpec((1, Nout), lambda i: (0, 0)),
        ],
        out_specs=pl.BlockSpec((blk, Nout), lambda i: (i, 0)),
        out_shape=jax.ShapeDtypeStruct((NPAD, Nout), jnp.float32),
    )(acc_p, den_p, bias.reshape(1, -1), W, b2.reshape(1, -1))


def _final_kernel(acc_ref, den_ref, bias_ref, o_ref):
    den = den_ref[0] + den_ref[1]
    inv = 1.0 / (den + 1e-16)
    o_ref[...] = (acc_ref[0] + acc_ref[1]) * inv[:, None] + bias_ref[...]


def _final_combine(acc_p, den_p, bias, blk=2048):
    D = acc_p.shape[2]
    return pl.pallas_call(
        _final_kernel,
        grid=(NPAD // blk,),
        in_specs=[
            pl.BlockSpec((2, blk, D), lambda i: (0, i, 0)),
            pl.BlockSpec((2, blk), lambda i: (0, i)),
            pl.BlockSpec((1, D), lambda i: (0, 0)),
        ],
        out_specs=pl.BlockSpec((blk, D), lambda i: (i, 0)),
        out_shape=jax.ShapeDtypeStruct((NPAD, D), jnp.float32),
    )(acc_p, den_p, bias.reshape(1, -1))


def _make_gat_sc(D):
    mesh = plsc.VectorSubcoreMesh(
        core_axis_name="c", subcore_axis_name="s", num_cores=NC, num_subcores=NS
    )
    CH = D // 16
    RPT = NPAD // NS
    ZR = 128

    def body(xl_hbm, src_hbm, dst_hbm, ef_hbm, att_hbm,
             den_out, acc_out,
             src_v, dst_v, xs_v, xd_v, ef_v, ex_v, att_v, zrow_v, zden_v,
             den_s, acc_s, sem0, sem1, sem2):
        c = lax.axis_index("c")
        s = lax.axis_index("s")
        g = c * NS + s

        zero16 = jnp.zeros((16,), jnp.float32)

        def zden_body(i, _):
            zden_v[pl.ds(i * 16, 16)] = zero16
            return 0
        lax.fori_loop(0, RPT // 16, zden_body, 0)

        def zrow_body(i, _):
            for cc in range(CH):
                zrow_v[i, pl.ds(cc * 16, 16)] = zero16
            return 0
        lax.fori_loop(0, ZR, zrow_body, 0)

        pltpu.sync_copy(zden_v, den_s.at[pl.ds(s * RPT, RPT)])
        for j in range(RPT // ZR):
            pltpu.sync_copy(zrow_v, acc_s.at[pl.ds(s * RPT + j * ZR, ZR)])
        plsc.subcore_barrier()

        pltpu.sync_copy(att_hbm, att_v)
        iota16 = lax.iota(jnp.int32, 16)

        def batch(t, _):
            base = g * EPW + t * B
            pltpu.sync_copy(src_hbm.at[pl.ds(base, B)], src_v)
            pltpu.sync_copy(dst_hbm.at[pl.ds(base, B)], dst_v)
            cp0 = pltpu.async_copy(xl_hbm.at[src_v], xs_v, sem0)
            cp1 = pltpu.async_copy(xl_hbm.at[dst_v], xd_v, sem1)
            cp2 = pltpu.async_copy(ef_hbm.at[pl.ds(base, B)], ef_v, sem2)
            cp0.wait()
            cp1.wait()
            cp2.wait()

            def group(gi, _):
                e0 = gi * 16
                eidx = e0 + iota16

                def chunk(cc, acc):
                    attc = att_v[pl.ds(cc * 16, 16)]
                    f0 = cc * 16
                    for jj in range(16):
                        colf = jnp.full((16,), f0 + jj, jnp.int32)
                        vs = plsc.load_gather(xs_v, [eidx, colf])
                        vd = plsc.load_gather(xd_v, [eidx, colf])
                        ve = plsc.load_gather(ef_v, [eidx, colf])
                        sv = vs + vd + ve
                        lv = jnp.maximum(sv, 0.2 * sv)
                        acc = acc + lv * attc[jj]
                    return acc
                acc = lax.fori_loop(0, CH, chunk, jnp.zeros((16,), jnp.float32))
                ex16 = jnp.exp(acc)
                ex_v[pl.ds(e0, 16)] = ex16

                def scale(el, _):
                    e = e0 + el
                    exb = plsc.load_gather(ex_v, [jnp.full((16,), e, jnp.int32)])
                    for cc in range(CH):
                        xs_v[e, pl.ds(cc * 16, 16)] = (
                            xs_v[e, pl.ds(cc * 16, 16)] * exb
                        )
                    return 0
                lax.fori_loop(0, 16, scale, 0)
                return 0
            lax.fori_loop(0, B // 16, group, 0)

            pltpu.sync_copy(ex_v, den_s.at[dst_v], add=True)
            pltpu.sync_copy(xs_v, acc_s.at[dst_v], add=True)
            return 0
        lax.fori_loop(0, NB, batch, 0)

        plsc.subcore_barrier()
        pltpu.sync_copy(den_s.at[pl.ds(s * RPT, RPT)],
                        den_out.at[c, pl.ds(s * RPT, RPT)])
        pltpu.sync_copy(acc_s.at[pl.ds(s * RPT, RPT)],
                        acc_out.at[c, pl.ds(s * RPT, RPT)])

    return pl.kernel(
        body,
        out_type=(
            jax.ShapeDtypeStruct((NC, NPAD), jnp.float32),
            jax.ShapeDtypeStruct((NC, NPAD, D), jnp.float32),
        ),
        mesh=mesh,
        compiler_params=pltpu.CompilerParams(
            needs_layout_passes=False, use_tc_tiling_on_sc=False
        ),
        scratch_types=[
            pltpu.VMEM((B,), jnp.int32),
            pltpu.VMEM((B,), jnp.int32),
            pltpu.VMEM((B, D), jnp.float32),
            pltpu.VMEM((B, D), jnp.float32),
            pltpu.VMEM((B, D), jnp.float32),
            pltpu.VMEM((B,), jnp.float32),
            pltpu.VMEM((D,), jnp.float32),
            pltpu.VMEM((ZR, D), jnp.float32),
            pltpu.VMEM((RPT,), jnp.float32),
            pltpu.VMEM_SHARED((NPAD,), jnp.float32),
            pltpu.VMEM_SHARED((NPAD, D), jnp.float32),
            pltpu.SemaphoreType.DMA,
            pltpu.SemaphoreType.DMA,
            pltpu.SemaphoreType.DMA,
        ],
    )


_gat_sc_128 = _make_gat_sc(LATENT)
_gat_sc_16 = _make_gat_sc(N_ACT)


def kernel(x, edge_index, edge_attr,
           W1, b1, We1, att1, bias1,
           W2, b2, We2, att2, bias2):
    src = edge_index[0]
    dst = edge_index[1]
    zero128 = jnp.zeros((LATENT,), jnp.float32)
    zero16 = jnp.zeros((N_ACT,), jnp.float32)

    xl1 = _mm_bias(x, W1, b1, blk=2000)
    ef1 = _mm_bias(edge_attr, We1, zero128, blk=4000)
    den1, acc1 = _gat_sc_128(xl1, src, dst, ef1, att1)

    xl2 = _combine_mm(acc1, den1, bias1, W2, b2)
    ef2 = _mm_bias(edge_attr, We2, zero16, blk=4000)
    den2, acc2 = _gat_sc_16(xl2, src, dst, ef2, att2)

    action_logits = _final_combine(acc2, den2, bias2)[:N]

    flat = action_logits.reshape(-1)
    skey = jax.random.key(42)
    idx = jax.random.categorical(skey, flat)
    log_prob = jax.nn.log_softmax(flat)[idx]
    sel_node, sel_action = jnp.unravel_index(idx, action_logits.shape)
    return (sel_node, sel_action, log_prob)
